# baseline (speedup 1.0000x reference)
import functools

import jax
import jax.numpy as jnp
from jax import lax
from jax.experimental import pallas as pl
from jax.experimental.pallas import tpu as pltpu
from jax.experimental.pallas import tpu_sc as plsc

NC = 2
NS = 16
NW = NC * NS

CHUNK = 128
RING = 4
LOOKAHEAD = 2


def _make_sc_call(n_chunks, seq, embed, vocab):
    cpw = n_chunks // NW
    assert cpw % RING == 0
    pe_rows = seq + CHUNK

    def body(idx_hbm, table_hbm, pe_hbm, out_hbm, idx_v, pe_v,
             r0, r1, r2, r3, g0, g1, g2, g3, s0, s1, s2, s3):
        rows = (r0, r1, r2, r3)
        gsem = (g0, g1, g2, g3)
        ssem = (s0, s1, s2, s3)
        cid = lax.axis_index("c")
        sid = lax.axis_index("s")
        wid = sid * NC + cid
        base = wid * cpw

        pltpu.sync_copy(idx_hbm.at[pl.ds(base, cpw)], idx_v)
        pltpu.sync_copy(pe_hbm, pe_v)

        def gather(t, b):
            return pltpu.make_async_copy(
                table_hbm.at[idx_v.at[t]], rows[b], gsem[b])

        def store(t, b):
            return pltpu.make_async_copy(
                rows[b], out_hbm.at[pl.ds((base + t) * CHUNK, CHUNK)], ssem[b])

        gather(0, 0).start()
        gather(1, 1).start()

        nvec = embed // 16

        def outer(t0, carry):
            for b in range(RING):
                t = t0 * RING + b
                gather(t, b).wait()
                s_start = lax.rem(t * CHUNK, seq)

                def row_body(r, _, b=b, s_start=s_start):
                    sr = s_start + r
                    for k in range(nvec):
                        sl = pl.ds(k * 16, 16)
                        rows[b][r, sl] = rows[b][r, sl] + pe_v[sr, sl]
                    return 0

                lax.fori_loop(0, CHUNK, row_body, 0, unroll=4)
                store(t, b).start()

                tn = t + LOOKAHEAD
                bn = (b + LOOKAHEAD) % RING

                @pl.when(tn < cpw)
                def _(tn=tn, bn=bn):
                    @pl.when(tn >= RING)
                    def _():
                        store(tn - RING, bn).wait()
                    gather(tn, bn).start()
            return carry

        lax.fori_loop(0, cpw // RING, outer, 0)

        for b in range(RING):
            store(cpw - RING + b, b).wait()

    return pl.kernel(
        body,
        out_type=jax.ShapeDtypeStruct((n_chunks * CHUNK, embed), jnp.float32),
        mesh=plsc.VectorSubcoreMesh(core_axis_name="c", subcore_axis_name="s"),
        compiler_params=pltpu.CompilerParams(use_tc_tiling_on_sc=False),
        scratch_types=[
            pltpu.VMEM((cpw, CHUNK), jnp.int32),
            pltpu.VMEM((pe_rows, embed), jnp.float32),
            pltpu.VMEM((CHUNK, embed), jnp.float32),
            pltpu.VMEM((CHUNK, embed), jnp.float32),
            pltpu.VMEM((CHUNK, embed), jnp.float32),
            pltpu.VMEM((CHUNK, embed), jnp.float32),
            pltpu.SemaphoreType.DMA,
            pltpu.SemaphoreType.DMA,
            pltpu.SemaphoreType.DMA,
            pltpu.SemaphoreType.DMA,
            pltpu.SemaphoreType.DMA,
            pltpu.SemaphoreType.DMA,
            pltpu.SemaphoreType.DMA,
            pltpu.SemaphoreType.DMA,
        ],
    )


def _pos_encoding(seq_len, d_model):
    pos = jnp.arange(1, 1 + seq_len, dtype=jnp.float32)
    power = jnp.arange(0, d_model, 2, dtype=jnp.float32) / d_model
    divisor = jnp.power(10000.0, power)
    angles = pos[:, None] / divisor[None, :]
    return jnp.stack([jnp.sin(angles), jnp.cos(angles)], axis=-1).reshape(
        seq_len, d_model)


@functools.partial(jax.jit, static_argnames=())
def kernel(inputs, table):
    batch, seq = inputs.shape
    vocab, embed = table.shape
    n = batch * seq
    assert n % (NW * CHUNK) == 0
    n_chunks = n // CHUNK

    pe = _pos_encoding(seq, embed)
    pe_ext = jnp.concatenate([pe, pe[:CHUNK]], axis=0)
    idx = inputs.reshape(n_chunks, CHUNK)

    call = _make_sc_call(n_chunks, seq, embed, vocab)
    out = call(idx, table, pe_ext)
    return out.reshape(batch, seq, embed)
